# baseline (speedup 1.0000x reference)
import functools

import jax
import jax.numpy as jnp
from jax import lax
from jax.experimental import pallas as pl
from jax.experimental.pallas import tpu as pltpu
from jax.experimental.pallas import tpu_sc as plsc

_DROP_ROWS = (7, 6, 3, 2)


def _sc_body(x, z, o, sem):
    cc = lax.axis_index("c")
    ss = lax.axis_index("s")
    wid = ss * 2 + cc

    dmas = []
    for k in range(3):
        cid = wid * 3 + k
        kr = cid // 8
        row = kr + jnp.where(kr >= 2, 2, 0) + jnp.where(kr >= 4, 2, 0)
        t0 = (cid % 8) * 256
        d = pltpu.make_async_copy(
            x.at[row, pl.ds(t0, 256)], o.at[row, pl.ds(t0, 256)], sem.at[k])
        d.start()
        dmas.append(d)

    q = wid // 8
    m = wid % 8
    zrow = 2 + (q % 2) + 4 * (q // 2)
    t0 = m * 256

    @pl.when(m < 7)
    def _():
        d = pltpu.make_async_copy(
            z.at[pl.ds(0, 256)], o.at[zrow, pl.ds(t0, 256)], sem.at[3])
        d.start()
        d.wait()

    @pl.when(m == 7)
    def _():
        d = pltpu.make_async_copy(
            z.at[pl.ds(0, 255)], o.at[zrow, pl.ds(t0, 255)], sem.at[4])
        d.start()
        d.wait()
        s = pltpu.make_async_copy(
            x.at[zrow, pl.ds(2047, 1)], o.at[zrow, pl.ds(2047, 1)], sem.at[5])
        s.start()
        s.wait()

    for d in dmas:
        d.wait()


def kernel(emb0):
    b, t, c, d = emb0.shape
    z = jnp.zeros((256, c, d), emb0.dtype)
    mesh = plsc.VectorSubcoreMesh(core_axis_name="c", subcore_axis_name="s")
    run = functools.partial(
        pl.kernel,
        mesh=mesh,
        out_type=jax.ShapeDtypeStruct((b, t, c, d), emb0.dtype),
        scratch_types=[pltpu.SemaphoreType.DMA((8,))],
    )(_sc_body)
    return run(emb0, z)


# speedup vs baseline: 31.0976x; 31.0976x over previous
import functools

import jax
import jax.numpy as jnp
from jax import lax
from jax.experimental import pallas as pl
from jax.experimental.pallas import tpu as pltpu
from jax.experimental.pallas import tpu_sc as plsc

_DROP_ROWS = (7, 6, 3, 2)

_CHUNK = 64
_NCH = 12
_NBUF = 2


def _sc_body(x, z, o, buf, zbuf, sem_in, sem_out, sem_z):
    cc = lax.axis_index("c")
    ss = lax.axis_index("s")
    wid = ss * 2 + cc
    cp = pltpu.make_async_copy

    zin = cp(z, zbuf, sem_z.at[0])
    zin.start()

    locs = []
    for k in range(_NCH):
        cid = wid * _NCH + k
        kr = cid // 32
        row = kr + jnp.where(kr >= 2, 2, 0) + jnp.where(kr >= 4, 2, 0)
        t0 = (cid % 32) * _CHUNK
        locs.append((row, t0))

    q = wid // 8
    m = wid % 8
    zrow = 2 + (q % 2) + 4 * (q // 2)
    zt0 = m * 256
    zin.wait()

    @pl.when(m < 7)
    def _():
        for j in range(4):
            cp(zbuf, o.at[zrow, pl.ds(zt0 + j * _CHUNK, _CHUNK)],
               sem_z.at[1 + j]).start()

    @pl.when(m == 7)
    def _():
        for j in range(3):
            cp(zbuf, o.at[zrow, pl.ds(zt0 + j * _CHUNK, _CHUNK)],
               sem_z.at[1 + j]).start()
        cp(zbuf.at[pl.ds(0, _CHUNK - 1)],
           o.at[zrow, pl.ds(zt0 + 3 * _CHUNK, _CHUNK - 1)],
           sem_z.at[4]).start()
        cp(x.at[zrow, pl.ds(2047, 1)], o.at[zrow, pl.ds(2047, 1)],
           sem_z.at[5]).start()

    ins = [None] * _NCH
    outs = [None] * _NCH

    def start_in(k):
        row, t0 = locs[k]
        d = cp(x.at[row, pl.ds(t0, _CHUNK)], buf.at[k % _NBUF],
               sem_in.at[k % _NBUF])
        d.start()
        ins[k] = d

    start_in(0)
    for k in range(_NCH):
        if k + 1 < _NCH:
            if k - 1 >= 0:
                outs[k - 1].wait()
            start_in(k + 1)
        ins[k].wait()
        row, t0 = locs[k]
        d = cp(buf.at[k % _NBUF], o.at[row, pl.ds(t0, _CHUNK)],
               sem_out.at[k % _NBUF])
        d.start()
        outs[k] = d

    outs[_NCH - 2].wait()
    outs[_NCH - 1].wait()

    @pl.when(m < 7)
    def _():
        for j in range(4):
            cp(zbuf, o.at[zrow, pl.ds(zt0 + j * _CHUNK, _CHUNK)],
               sem_z.at[1 + j]).wait()

    @pl.when(m == 7)
    def _():
        for j in range(3):
            cp(zbuf, o.at[zrow, pl.ds(zt0 + j * _CHUNK, _CHUNK)],
               sem_z.at[1 + j]).wait()
        cp(zbuf.at[pl.ds(0, _CHUNK - 1)],
           o.at[zrow, pl.ds(zt0 + 3 * _CHUNK, _CHUNK - 1)],
           sem_z.at[4]).wait()
        cp(x.at[zrow, pl.ds(2047, 1)], o.at[zrow, pl.ds(2047, 1)],
           sem_z.at[5]).wait()


def kernel(emb0):
    b, t, c, d = emb0.shape
    z = jnp.zeros((_CHUNK, c, d), emb0.dtype)
    mesh = plsc.VectorSubcoreMesh(core_axis_name="c", subcore_axis_name="s")
    run = functools.partial(
        pl.kernel,
        mesh=mesh,
        out_type=jax.ShapeDtypeStruct((b, t, c, d), emb0.dtype),
        scratch_types=[
            pltpu.VMEM((_NBUF, _CHUNK, c, d), emb0.dtype),
            pltpu.VMEM((_CHUNK, c, d), emb0.dtype),
            pltpu.SemaphoreType.DMA((_NBUF,)),
            pltpu.SemaphoreType.DMA((_NBUF,)),
            pltpu.SemaphoreType.DMA((6,)),
        ],
    )(_sc_body)
    return run(emb0, z)


# speedup vs baseline: 32.1011x; 1.0323x over previous
import functools

import jax
import jax.numpy as jnp
from jax import lax
from jax.experimental import pallas as pl
from jax.experimental.pallas import tpu as pltpu
from jax.experimental.pallas import tpu_sc as plsc

_DROP_ROWS = (7, 6, 3, 2)

_CHUNK = 64
_NCH = 12
_NBUF = 3


def _sc_body(x, z, o, buf, zbuf, sem_in, sem_out, sem_z):
    cc = lax.axis_index("c")
    ss = lax.axis_index("s")
    wid = ss * 2 + cc
    cp = pltpu.make_async_copy

    zin = cp(z, zbuf, sem_z.at[0])
    zin.start()

    locs = []
    for k in range(_NCH):
        cid = wid * _NCH + k
        kr = cid // 32
        row = kr + jnp.where(kr >= 2, 2, 0) + jnp.where(kr >= 4, 2, 0)
        t0 = (cid % 32) * _CHUNK
        locs.append((row, t0))

    q = wid // 8
    m = wid % 8
    zrow = 2 + (q % 2) + 4 * (q // 2)
    zt0 = m * 256
    zin.wait()

    @pl.when(m < 7)
    def _():
        for j in range(8):
            cp(zbuf, o.at[zrow, pl.ds(zt0 + j * 32, 32)],
               sem_z.at[1 + j]).start()

    @pl.when(m == 7)
    def _():
        for j in range(7):
            cp(zbuf, o.at[zrow, pl.ds(zt0 + j * 32, 32)],
               sem_z.at[1 + j]).start()
        cp(zbuf.at[pl.ds(0, 31)],
           o.at[zrow, pl.ds(zt0 + 7 * 32, 31)], sem_z.at[8]).start()
        cp(x.at[zrow, pl.ds(2047, 1)], o.at[zrow, pl.ds(2047, 1)],
           sem_z.at[9]).start()

    ins = [None] * _NCH
    outs = [None] * _NCH

    def start_in(k):
        row, t0 = locs[k]
        d = cp(x.at[row, pl.ds(t0, _CHUNK)], buf.at[k % _NBUF],
               sem_in.at[k % _NBUF])
        d.start()
        ins[k] = d

    start_in(0)
    start_in(1)
    for k in range(_NCH):
        if k + 2 < _NCH:
            if k - 1 >= 0:
                outs[k - 1].wait()
            start_in(k + 2)
        ins[k].wait()
        row, t0 = locs[k]
        d = cp(buf.at[k % _NBUF], o.at[row, pl.ds(t0, _CHUNK)],
               sem_out.at[k % _NBUF])
        d.start()
        outs[k] = d

    outs[_NCH - 2].wait()
    outs[_NCH - 1].wait()

    @pl.when(m < 7)
    def _():
        for j in range(8):
            cp(zbuf, o.at[zrow, pl.ds(zt0 + j * 32, 32)],
               sem_z.at[1 + j]).wait()

    @pl.when(m == 7)
    def _():
        for j in range(7):
            cp(zbuf, o.at[zrow, pl.ds(zt0 + j * 32, 32)],
               sem_z.at[1 + j]).wait()
        cp(zbuf.at[pl.ds(0, 31)],
           o.at[zrow, pl.ds(zt0 + 7 * 32, 31)], sem_z.at[8]).wait()
        cp(x.at[zrow, pl.ds(2047, 1)], o.at[zrow, pl.ds(2047, 1)],
           sem_z.at[9]).wait()


def kernel(emb0):
    b, t, c, d = emb0.shape
    z = jnp.zeros((32, c, d), emb0.dtype)
    mesh = plsc.VectorSubcoreMesh(core_axis_name="c", subcore_axis_name="s")
    run = functools.partial(
        pl.kernel,
        mesh=mesh,
        out_type=jax.ShapeDtypeStruct((b, t, c, d), emb0.dtype),
        scratch_types=[
            pltpu.VMEM((_NBUF, _CHUNK, c, d), emb0.dtype),
            pltpu.VMEM((32, c, d), emb0.dtype),
            pltpu.SemaphoreType.DMA((_NBUF,)),
            pltpu.SemaphoreType.DMA((_NBUF,)),
            pltpu.SemaphoreType.DMA((10,)),
        ],
    )(_sc_body)
    return run(emb0, z)


# speedup vs baseline: 56.3889x; 1.7566x over previous
import functools

import jax
import jax.numpy as jnp
from jax.experimental import pallas as pl
from jax.experimental.pallas import tpu as pltpu

_DROP_ROWS = (7, 6, 3, 2)

_TC = 2048
_K = 6
_D = 3


def _body(x, o, ring, zbuf, tbuf, sem_in, sem_out, sem_z, sem_t, *, b, t, c, d):
    cp = pltpu.make_async_copy
    dropped = sorted(_DROP_ROWS)
    kept = [i for i in range(b) if i not in dropped]
    jobs = [(i, t0) for i in kept for t0 in range(0, t, _TC)]
    n = len(jobs)

    in_dma = [None] * n
    out_dma = [None] * n

    def start_in(m):
        i, t0 = jobs[m]
        dcp = cp(x.at[i, pl.ds(t0, _TC)], ring.at[m % _K], sem_in.at[m % _K])
        dcp.start()
        in_dma[m] = dcp

    def start_out(m):
        i, t0 = jobs[m]
        dcp = cp(ring.at[m % _K], o.at[i, pl.ds(t0, _TC)], sem_out.at[m % _K])
        dcp.start()
        out_dma[m] = dcp

    for m in range(_D):
        start_in(m)

    zbuf[...] = jnp.zeros_like(zbuf)
    zq = []
    for q, row in enumerate(dropped):
        d0 = cp(zbuf, o.at[row, pl.ds(0, t - 1)], sem_z.at[q])
        d0.start()
        zq.append(d0)

    tin = []
    for q, row in enumerate(dropped):
        dcp = cp(x.at[row, pl.ds(t - 1, 1)], tbuf.at[q], sem_t.at[q])
        dcp.start()
        tin.append(dcp)

    for m in range(n + _D):
        if m < n and m >= _D:
            if m - _K >= 0:
                out_dma[m - _K].wait()
            start_in(m)
        j = m - _D
        if 0 <= j < n:
            in_dma[j].wait()
            start_out(j)

    for q, row in enumerate(dropped):
        tin[q].wait()
        dcp = cp(tbuf.at[q], o.at[row, pl.ds(t - 1, 1)], sem_t.at[4 + q])
        dcp.start()
        zq.append(dcp)

    for j in range(max(0, n - _K), n):
        out_dma[j].wait()
    for dcp in zq:
        dcp.wait()


def kernel(emb0):
    b, t, c, d = emb0.shape
    zrows = t - 1

    return pl.pallas_call(
        functools.partial(_body, b=b, t=t, c=c, d=d),
        in_specs=[pl.BlockSpec(memory_space=pl.ANY)],
        out_specs=pl.BlockSpec(memory_space=pl.ANY),
        out_shape=jax.ShapeDtypeStruct((b, t, c, d), emb0.dtype),
        scratch_shapes=[
            pltpu.VMEM((_K, _TC, c, d), emb0.dtype),
            pltpu.VMEM((zrows, c, d), emb0.dtype),
            pltpu.VMEM((len(_DROP_ROWS), 1, c, d), emb0.dtype),
            pltpu.SemaphoreType.DMA((_K,)),
            pltpu.SemaphoreType.DMA((_K,)),
            pltpu.SemaphoreType.DMA((2 * len(_DROP_ROWS),)),
            pltpu.SemaphoreType.DMA((2 * len(_DROP_ROWS),)),
        ],
    )(emb0)
